# Initial kernel scaffold; baseline (speedup 1.0000x reference)
import jax
import jax.numpy as jnp
from jax.experimental import pallas as pl


def kernel(y_pred, y_true, weights, hist_bricks):
    raise NotImplementedError("write your pallas kernel here")


# speedup vs baseline: 6.6262x; 6.6262x over previous
import functools

import jax
import jax.numpy as jnp
from jax import lax
from jax.experimental import pallas as pl
from jax.experimental.pallas import tpu as pltpu
from jax.experimental.pallas import tpu_sc as plsc

_NC = 2
_NS = 16
_L = 16
_NW = _NC * _NS


def _sc_hist_body(cp, chunk, num_classes,
                  y_pred_hbm, y_true_hbm, counts_out, soft_out, o_out,
                  yp_v, yt_v, cnt_rows, soft_rows, cnt_red, soft_red, o_red):
    wid = lax.axis_index("s") * _NC + lax.axis_index("c")
    base = wid * chunk

    pltpu.sync_copy(y_pred_hbm.at[pl.ds(base, chunk)], yp_v)
    pltpu.sync_copy(y_true_hbm.at[pl.ds(base, chunk)], yt_v)

    zeros16 = jnp.zeros((_L,), jnp.float32)
    ones16 = jnp.ones((_L,), jnp.float32)
    lane = lax.iota(jnp.int32, _L)

    def zero_body(i, c):
        r = i // (cp // _L)
        j = i % (cp // _L)
        cnt_rows[r, pl.ds(j * _L, _L)] = zeros16
        soft_rows[r, pl.ds(j * _L, _L)] = zeros16
        return c
    lax.fori_loop(0, _L * (cp // _L), zero_body, 0)

    def hist_body(g, o_acc):
        yp = yp_v[pl.ds(g * _L, _L)]
        yt = yt_v[pl.ds(g * _L, _L)]
        d = yp - yt.astype(jnp.float32)
        o_acc = o_acc + d * d
        plsc.addupdate_scatter(cnt_rows, [lane, yt], ones16)
        ypc = jnp.clip(yp, 0.0, float(num_classes - 1))
        f = ypc.astype(jnp.int32)
        p = ypc - f.astype(jnp.float32)
        cl = f + (p > 0.0).astype(jnp.int32)
        plsc.addupdate_scatter(soft_rows, [lane, f], ones16 - p)
        plsc.addupdate_scatter(soft_rows, [lane, cl], p)
        return o_acc
    o_acc = lax.fori_loop(0, chunk // _L, hist_body,
                          jnp.zeros((_L,), jnp.float32))
    o_red[...] = o_acc

    def red_body(j, c):
        ca = cnt_rows[0, pl.ds(j * _L, _L)]
        sa = soft_rows[0, pl.ds(j * _L, _L)]
        for r in range(1, _L):
            ca = ca + cnt_rows[r, pl.ds(j * _L, _L)]
            sa = sa + soft_rows[r, pl.ds(j * _L, _L)]
        cnt_red[pl.ds(j * _L, _L)] = ca
        soft_red[pl.ds(j * _L, _L)] = sa
        return c
    lax.fori_loop(0, cp // _L, red_body, 0)

    pltpu.sync_copy(cnt_red, counts_out.at[wid])
    pltpu.sync_copy(soft_red, soft_out.at[wid])
    pltpu.sync_copy(o_red, o_out.at[pl.ds(wid * _L, _L)])


def _make_sc_hist(batch, cp, num_classes):
    chunk = batch // _NW
    mesh = plsc.VectorSubcoreMesh(core_axis_name="c", subcore_axis_name="s")
    return functools.partial(
        pl.kernel,
        mesh=mesh,
        compiler_params=pltpu.CompilerParams(use_tc_tiling_on_sc=False,
                                             needs_layout_passes=False),
        out_type=(
            jax.ShapeDtypeStruct((_NW, cp), jnp.float32),
            jax.ShapeDtypeStruct((_NW, cp), jnp.float32),
            jax.ShapeDtypeStruct((_NW * _L,), jnp.float32),
        ),
        scratch_types=[
            pltpu.VMEM((chunk,), jnp.float32),
            pltpu.VMEM((chunk,), jnp.int32),
            pltpu.VMEM((_L, cp), jnp.float32),
            pltpu.VMEM((_L, cp), jnp.float32),
            pltpu.VMEM((cp,), jnp.float32),
            pltpu.VMEM((cp,), jnp.float32),
            pltpu.VMEM((_L,), jnp.float32),
        ],
    )(functools.partial(_sc_hist_body, cp, chunk, num_classes))


def _combine_body(batch, num_classes, eps,
                  counts_ref, soft_ref, o_ref, hb_ref, w_ref, out_ref):
    counts = jnp.sum(counts_ref[...], axis=0, keepdims=True)
    soft = jnp.sum(soft_ref[...], axis=0, keepdims=True)
    o_total = jnp.sum(o_ref[...])
    hi = jax.lax.Precision.HIGHEST
    ht = jnp.dot(counts[:, :num_classes], hb_ref[...], precision=hi,
                 preferred_element_type=jnp.float32)
    hp = jnp.dot(soft[:, :num_classes], hb_ref[...], precision=hi,
                 preferred_element_type=jnp.float32)
    t = jnp.dot(ht, w_ref[...], precision=hi,
                preferred_element_type=jnp.float32)
    e = jnp.sum(t * hp) / float(batch)
    out_ref[...] = jnp.log(o_total / (e + eps)).reshape(1, 1)


def _combine(counts_p, soft_p, o_p, hist_bricks, weights, batch, eps):
    num_classes = hist_bricks.shape[0]
    body = functools.partial(_combine_body, batch, num_classes, eps)
    out = pl.pallas_call(
        body,
        out_shape=jax.ShapeDtypeStruct((1, 1), jnp.float32),
    )(counts_p, soft_p, o_p, hist_bricks, weights)
    return out[0, 0]


def kernel(y_pred, y_true, weights, hist_bricks):
    batch = y_pred.shape[0]
    num_classes = hist_bricks.shape[0]
    cp = ((num_classes + 127) // 128) * 128

    ypf = y_pred.reshape(batch)
    yti = y_true.reshape(batch).astype(jnp.int32)

    counts_p, soft_p, o_p = _make_sc_hist(batch, cp, num_classes)(ypf, yti)
    o_p = o_p.reshape(4, (_NW * _L) // 4)
    return _combine(counts_p, soft_p, o_p, hist_bricks, weights,
                    batch, 1e-10)


# speedup vs baseline: 7.0540x; 1.0646x over previous
import functools

import jax
import jax.numpy as jnp
from jax import lax
from jax.experimental import pallas as pl
from jax.experimental.pallas import tpu as pltpu
from jax.experimental.pallas import tpu_sc as plsc

_NC = 2
_NS = 16
_L = 16
_NW = _NC * _NS


def _sc_hist_body(cp, chunk, num_classes,
                  y_pred_hbm, y_true_hbm, counts_out, soft_out, o_out,
                  yp_v, yt_v, cnt_rows, soft_rows, cnt_red, soft_red, o_red):
    wid = lax.axis_index("s") * _NC + lax.axis_index("c")
    base = wid * chunk

    pltpu.sync_copy(y_pred_hbm.at[pl.ds(base, chunk)], yp_v)
    pltpu.sync_copy(y_true_hbm.at[pl.ds(base, chunk)], yt_v)

    zeros16 = jnp.zeros((_L,), jnp.float32)
    ones16 = jnp.ones((_L,), jnp.float32)
    lane = lax.iota(jnp.int32, _L)

    def zero_body(j, c):
        for r in range(_L):
            cnt_rows[r, pl.ds(j * _L, _L)] = zeros16
            soft_rows[r, pl.ds(j * _L, _L)] = zeros16
        return c
    lax.fori_loop(0, cp // _L, zero_body, 0)

    def hist_body(g, o_acc):
        yp = yp_v[pl.ds(g * _L, _L)]
        yt = yt_v[pl.ds(g * _L, _L)]
        d = yp - yt.astype(jnp.float32)
        o_acc = o_acc + d * d
        plsc.addupdate_scatter(cnt_rows, [lane, yt], ones16)
        ypc = jnp.clip(yp, 0.0, float(num_classes - 1))
        f = ypc.astype(jnp.int32)
        p = ypc - f.astype(jnp.float32)
        cl = f + (p > 0.0).astype(jnp.int32)
        plsc.addupdate_scatter(soft_rows, [lane, f], ones16 - p)
        plsc.addupdate_scatter(soft_rows, [lane, cl], p)
        return o_acc
    o_acc = lax.fori_loop(0, chunk // _L, hist_body,
                          jnp.zeros((_L,), jnp.float32))
    o_red[...] = o_acc

    def red_body(j, c):
        ca = cnt_rows[0, pl.ds(j * _L, _L)]
        sa = soft_rows[0, pl.ds(j * _L, _L)]
        for r in range(1, _L):
            ca = ca + cnt_rows[r, pl.ds(j * _L, _L)]
            sa = sa + soft_rows[r, pl.ds(j * _L, _L)]
        cnt_red[pl.ds(j * _L, _L)] = ca
        soft_red[pl.ds(j * _L, _L)] = sa
        return c
    lax.fori_loop(0, cp // _L, red_body, 0)

    pltpu.sync_copy(cnt_red, counts_out.at[wid])
    pltpu.sync_copy(soft_red, soft_out.at[wid])
    pltpu.sync_copy(o_red, o_out.at[pl.ds(wid * _L, _L)])


def _make_sc_hist(batch, cp, num_classes):
    chunk = batch // _NW
    mesh = plsc.VectorSubcoreMesh(core_axis_name="c", subcore_axis_name="s")
    return functools.partial(
        pl.kernel,
        mesh=mesh,
        compiler_params=pltpu.CompilerParams(use_tc_tiling_on_sc=False,
                                             needs_layout_passes=False),
        out_type=(
            jax.ShapeDtypeStruct((_NW, cp), jnp.float32),
            jax.ShapeDtypeStruct((_NW, cp), jnp.float32),
            jax.ShapeDtypeStruct((_NW * _L,), jnp.float32),
        ),
        scratch_types=[
            pltpu.VMEM((chunk,), jnp.float32),
            pltpu.VMEM((chunk,), jnp.int32),
            pltpu.VMEM((_L, cp), jnp.float32),
            pltpu.VMEM((_L, cp), jnp.float32),
            pltpu.VMEM((cp,), jnp.float32),
            pltpu.VMEM((cp,), jnp.float32),
            pltpu.VMEM((_L,), jnp.float32),
        ],
    )(functools.partial(_sc_hist_body, cp, chunk, num_classes))


def _combine_body(batch, num_classes, eps,
                  counts_ref, soft_ref, o_ref, hb_ref, w_ref, out_ref):
    counts = jnp.sum(counts_ref[...], axis=0, keepdims=True)
    soft = jnp.sum(soft_ref[...], axis=0, keepdims=True)
    o_total = jnp.sum(o_ref[...])
    hi = jax.lax.Precision.HIGHEST
    ht = jnp.dot(counts[:, :num_classes], hb_ref[...], precision=hi,
                 preferred_element_type=jnp.float32)
    hp = jnp.dot(soft[:, :num_classes], hb_ref[...], precision=hi,
                 preferred_element_type=jnp.float32)
    t = jnp.dot(ht, w_ref[...], precision=hi,
                preferred_element_type=jnp.float32)
    e = jnp.sum(t * hp) / float(batch)
    out_ref[...] = jnp.log(o_total / (e + eps)).reshape(1, 1)


def _combine(counts_p, soft_p, o_p, hist_bricks, weights, batch, eps):
    num_classes = hist_bricks.shape[0]
    body = functools.partial(_combine_body, batch, num_classes, eps)
    out = pl.pallas_call(
        body,
        out_shape=jax.ShapeDtypeStruct((1, 1), jnp.float32),
    )(counts_p, soft_p, o_p, hist_bricks, weights)
    return out[0, 0]


def kernel(y_pred, y_true, weights, hist_bricks):
    batch = y_pred.shape[0]
    num_classes = hist_bricks.shape[0]
    cp = ((num_classes + 127) // 128) * 128

    ypf = y_pred.reshape(batch)
    yti = y_true.reshape(batch).astype(jnp.int32)

    counts_p, soft_p, o_p = _make_sc_hist(batch, cp, num_classes)(ypf, yti)
    o_p = o_p.reshape(4, (_NW * _L) // 4)
    return _combine(counts_p, soft_p, o_p, hist_bricks, weights,
                    batch, 1e-10)
